# Initial kernel scaffold; baseline (speedup 1.0000x reference)
import functools

import jax
import jax.numpy as jnp
from jax import lax
from jax.experimental import pallas as pl
from jax.experimental.pallas import tpu as pltpu
from jax.experimental.pallas import tpu_sc as plsc

IN = 4096
HID = 4096
B = 64
RB = 64
DT = 0.1
NNZ = 167772

NC = 2
NS = 16
NW = NC * NS
CHUNK = 128
NCHUNK = -(-NNZ // (NW * CHUNK))
PER_W = NCHUNK * CHUNK
NNZ_PAD = NW * PER_W
ROWS_PER_TILE = HID // NS
LANES = 16

_MESH = plsc.VectorSubcoreMesh(core_axis_name="c", subcore_axis_name="s")


def _zero_contrib(contrib):
    zero16 = jnp.zeros((LANES,), jnp.float32)

    def zrow(i, _):
        for t in range(B // LANES):
            contrib[i, pl.ds(LANES * t, LANES)] = zero16
        return 0

    lax.fori_loop(0, CHUNK, zrow, 0)


def _accumulate(src_hbm, rows_hbm, cols_hbm, vals_hbm, acc, wid,
                rows_v, cols_v, vals_v, contrib, sem):
    pltpu.sync_copy(rows_hbm.at[wid], rows_v)
    pltpu.sync_copy(cols_hbm.at[wid], cols_v)
    pltpu.sync_copy(vals_hbm.at[wid], vals_v)

    def chunk_body(j, _):
        pltpu.async_copy(src_hbm.at[rows_v.at[j]], contrib, sem).wait()

        def srow(i, _):
            v = vals_v[j, i]
            for t in range(B // LANES):
                sl = pl.ds(LANES * t, LANES)
                contrib[i, sl] = contrib[i, sl] * v
            return 0

        lax.fori_loop(0, CHUNK, srow, 0, unroll=8)
        pltpu.sync_copy(contrib, acc.at[cols_v.at[j]], add=True)
        return 0

    lax.fori_loop(0, NCHUNK, chunk_body, 0)


def _spmm_sc_kernel(n_mats):
    def body(*refs):
        srcs = refs[0:n_mats]
        coo = refs[n_mats:4 * n_mats]
        outs = refs[4 * n_mats:5 * n_mats]
        accs = refs[5 * n_mats:6 * n_mats]
        rows_v, cols_v, vals_v, contrib, sem = refs[6 * n_mats:]

        cid = lax.axis_index("c")
        sid = lax.axis_index("s")
        wid = sid * NC + cid
        base = sid * ROWS_PER_TILE

        _zero_contrib(contrib)
        for m in range(n_mats):
            for h in range(ROWS_PER_TILE // CHUNK):
                pltpu.sync_copy(contrib, accs[m].at[pl.ds(base + h * CHUNK, CHUNK)])
        plsc.subcore_barrier()

        for m in range(n_mats):
            _accumulate(srcs[m], coo[3 * m], coo[3 * m + 1], coo[3 * m + 2],
                        accs[m], wid, rows_v, cols_v, vals_v, contrib, sem)
        plsc.subcore_barrier()

        for m in range(n_mats):
            pltpu.sync_copy(accs[m].at[pl.ds(base, ROWS_PER_TILE)],
                            outs[m].at[cid, pl.ds(base, ROWS_PER_TILE)])

    out_type = tuple(jax.ShapeDtypeStruct((NC, HID, B), jnp.float32)
                     for _ in range(n_mats))
    scratch = (
        [pltpu.MemorySpace.VMEM_SHARED((HID, B), jnp.float32) for _ in range(n_mats)]
        + [pltpu.VMEM((NCHUNK, CHUNK), jnp.int32),
           pltpu.VMEM((NCHUNK, CHUNK), jnp.int32),
           pltpu.VMEM((NCHUNK, CHUNK), jnp.float32),
           pltpu.VMEM((CHUNK, B), jnp.float32),
           pltpu.SemaphoreType.DMA]
    )
    return pl.kernel(body, out_type=out_type, mesh=_MESH, scratch_types=scratch)


_spmm2 = _spmm_sc_kernel(2)
_spmm1 = _spmm_sc_kernel(1)


def _fuse_body(xT, rW, rb, yW, yR, wb, rbias, hT, gT, tauT, out):
    rg = lax.dot_general(rW[...], xT[...], (((0,), (0,)), ((), ())),
                         preferred_element_type=jnp.float32)
    rg = jax.nn.sigmoid(rg + rb[...])
    mask = jnp.reshape(jnp.broadcast_to(rg[:, None, :], (RB, HID // RB, B)),
                       (HID, B))
    sensory = (yW[0] + yW[1] + wb[...]) * mask
    rec = yR[0] + yR[1] + rbias[...]
    target = jnp.tanh(sensory + rec)
    h = hT[...]
    out[...] = h + gT[...] * (target - h) * (DT / tauT[...])


_fuse = pl.pallas_call(
    _fuse_body,
    out_shape=jax.ShapeDtypeStruct((HID, B), jnp.float32),
)


def _combine_body(p, pb, out):
    out[...] = p[0] + p[1] + pb[...]


_combine = pl.pallas_call(
    _combine_body,
    out_shape=jax.ShapeDtypeStruct((HID, B), jnp.float32),
)


def _coo(rows, cols, vals):
    pad = NNZ_PAD - NNZ
    rows = jnp.pad(rows.astype(jnp.int32), (0, pad)).reshape(NW, NCHUNK, CHUNK)
    cols = jnp.pad(cols.astype(jnp.int32), (0, pad)).reshape(NW, NCHUNK, CHUNK)
    vals = jnp.pad(vals, (0, pad)).reshape(NW, NCHUNK, CHUNK)
    return rows, cols, vals


def kernel(x, h_prev, gate, W_rows, W_cols, W_vals, W_bias,
           R_rows, R_cols, R_vals, R_bias, P_rows, P_cols, P_vals, P_bias,
           router_W, router_b, tau):
    xT = x.T
    hT = h_prev.T
    gT = gate.T
    Wr, Wc, Wv = _coo(W_rows, W_cols, W_vals)
    Rr, Rc, Rv = _coo(R_rows, R_cols, R_vals)
    Pr, Pc, Pv = _coo(P_rows, P_cols, P_vals)

    yW, yR = _spmm2(xT, hT, Wr, Wc, Wv, Rr, Rc, Rv)
    h_newT = _fuse(xT, router_W, router_b.reshape(RB, 1), yW, yR,
                   W_bias.reshape(HID, 1), R_bias.reshape(HID, 1),
                   hT, gT, tau.reshape(HID, 1))
    (p,) = _spmm1(h_newT, Pr, Pc, Pv)
    predT = _combine(p, P_bias.reshape(HID, 1))
    return (h_newT.T, predT.T)


# speedup vs baseline: 6.2855x; 6.2855x over previous
import functools

import jax
import jax.numpy as jnp
from jax import lax
from jax.experimental import pallas as pl
from jax.experimental.pallas import tpu as pltpu
from jax.experimental.pallas import tpu_sc as plsc

IN = 4096
HID = 4096
B = 64
RB = 64
DT = 0.1
NNZ = 167772

NC = 2
NS = 16
NW = NC * NS
CHUNK = 128
NCHUNK = -(-NNZ // (NW * CHUNK))
PER_W = NCHUNK * CHUNK
NNZ_PAD = NW * PER_W
ROWS_PER_TILE = HID // NS
LANES = 16

_MESH = plsc.VectorSubcoreMesh(core_axis_name="c", subcore_axis_name="s")


def _zero_contrib(contrib):
    zero16 = jnp.zeros((LANES,), jnp.float32)

    def zrow(i, _):
        for t in range(B // LANES):
            contrib[i, pl.ds(LANES * t, LANES)] = zero16
        return 0

    lax.fori_loop(0, CHUNK, zrow, 0)


def _accumulate(src_hbm, rows_hbm, cols_hbm, vals_hbm, acc, wid,
                rows_v, cols_v, vals_v, contrib, sem):
    pltpu.sync_copy(rows_hbm.at[wid], rows_v)
    pltpu.sync_copy(cols_hbm.at[wid], cols_v)
    pltpu.sync_copy(vals_hbm.at[wid], vals_v)

    def chunk_body(j, _):
        pltpu.async_copy(src_hbm.at[rows_v.at[j]], contrib, sem).wait()

        def sgroup(g, _):
            vv = vals_v[j, pl.ds(LANES * g, LANES)]
            base_r = LANES * g
            for l in range(LANES):
                v = vv[l]
                for t in range(B // LANES):
                    sl = pl.ds(LANES * t, LANES)
                    contrib[base_r + l, sl] = contrib[base_r + l, sl] * v
            return 0

        lax.fori_loop(0, CHUNK // LANES, sgroup, 0)
        pltpu.sync_copy(contrib, acc.at[cols_v.at[j]], add=True)
        return 0

    lax.fori_loop(0, NCHUNK, chunk_body, 0)


def _spmm_sc_kernel(n_mats):
    def body(*refs):
        srcs = refs[0:n_mats]
        coo = refs[n_mats:4 * n_mats]
        outs = refs[4 * n_mats:5 * n_mats]
        accs = refs[5 * n_mats:6 * n_mats]
        rows_v, cols_v, vals_v, contrib, sem = refs[6 * n_mats:]

        cid = lax.axis_index("c")
        sid = lax.axis_index("s")
        wid = sid * NC + cid
        base = sid * ROWS_PER_TILE

        _zero_contrib(contrib)
        for m in range(n_mats):
            for h in range(ROWS_PER_TILE // CHUNK):
                pltpu.sync_copy(contrib, accs[m].at[pl.ds(base + h * CHUNK, CHUNK)])
        plsc.subcore_barrier()

        for m in range(n_mats):
            _accumulate(srcs[m], coo[3 * m], coo[3 * m + 1], coo[3 * m + 2],
                        accs[m], wid, rows_v, cols_v, vals_v, contrib, sem)
        plsc.subcore_barrier()

        for m in range(n_mats):
            pltpu.sync_copy(accs[m].at[pl.ds(base, ROWS_PER_TILE)],
                            outs[m].at[cid, pl.ds(base, ROWS_PER_TILE)])

    out_type = tuple(jax.ShapeDtypeStruct((NC, HID, B), jnp.float32)
                     for _ in range(n_mats))
    scratch = (
        [pltpu.MemorySpace.VMEM_SHARED((HID, B), jnp.float32) for _ in range(n_mats)]
        + [pltpu.VMEM((NCHUNK, CHUNK), jnp.int32),
           pltpu.VMEM((NCHUNK, CHUNK), jnp.int32),
           pltpu.VMEM((NCHUNK, CHUNK), jnp.float32),
           pltpu.VMEM((CHUNK, B), jnp.float32),
           pltpu.SemaphoreType.DMA]
    )
    return pl.kernel(body, out_type=out_type, mesh=_MESH, scratch_types=scratch,
                     compiler_params=pltpu.CompilerParams(use_tc_tiling_on_sc=False))


_spmm2 = _spmm_sc_kernel(2)
_spmm1 = _spmm_sc_kernel(1)


def _fuse_body(xT, rW, rb, yW, yR, wb, rbias, hT, gT, tauT, out):
    rg = lax.dot_general(rW[...], xT[...], (((0,), (0,)), ((), ())),
                         preferred_element_type=jnp.float32)
    rg = jax.nn.sigmoid(rg + rb[...])
    mask = jnp.reshape(jnp.broadcast_to(rg[:, None, :], (RB, HID // RB, B)),
                       (HID, B))
    sensory = (yW[0] + yW[1] + wb[...]) * mask
    rec = yR[0] + yR[1] + rbias[...]
    target = jnp.tanh(sensory + rec)
    h = hT[...]
    out[...] = h + gT[...] * (target - h) * (DT / tauT[...])


_fuse = pl.pallas_call(
    _fuse_body,
    out_shape=jax.ShapeDtypeStruct((HID, B), jnp.float32),
)


def _combine_body(p, pb, out):
    out[...] = p[0] + p[1] + pb[...]


_combine = pl.pallas_call(
    _combine_body,
    out_shape=jax.ShapeDtypeStruct((HID, B), jnp.float32),
)


def _coo(rows, cols, vals):
    pad = NNZ_PAD - NNZ
    rows = jnp.pad(rows.astype(jnp.int32), (0, pad)).reshape(NW, NCHUNK, CHUNK)
    cols = jnp.pad(cols.astype(jnp.int32), (0, pad)).reshape(NW, NCHUNK, CHUNK)
    vals = jnp.pad(vals, (0, pad)).reshape(NW, NCHUNK, CHUNK)
    return rows, cols, vals


def kernel(x, h_prev, gate, W_rows, W_cols, W_vals, W_bias,
           R_rows, R_cols, R_vals, R_bias, P_rows, P_cols, P_vals, P_bias,
           router_W, router_b, tau):
    xT = x.T
    hT = h_prev.T
    gT = gate.T
    Wr, Wc, Wv = _coo(W_rows, W_cols, W_vals)
    Rr, Rc, Rv = _coo(R_rows, R_cols, R_vals)
    Pr, Pc, Pv = _coo(P_rows, P_cols, P_vals)

    yW, yR = _spmm2(xT, hT, Wr, Wc, Wv, Rr, Rc, Rv)
    h_newT = _fuse(xT, router_W, router_b.reshape(RB, 1), yW, yR,
                   W_bias.reshape(HID, 1), R_bias.reshape(HID, 1),
                   hT, gT, tau.reshape(HID, 1))
    (p,) = _spmm1(h_newT, Pr, Pc, Pv)
    predT = _combine(p, P_bias.reshape(HID, 1))
    return (h_newT.T, predT.T)
